# baseline (speedup 1.0000x reference)
import functools
import jax
import jax.numpy as jnp
from jax import lax
from jax.experimental import pallas as pl
from jax.experimental.pallas import tpu as pltpu
from jax.experimental.pallas import tpu_sc as plsc

S, B, D = 4096, 4, 1024
NC, NS = 2, 16
NW = NC * NS
S_PER_W = S // NW
CH = 8
NCHUNK = S_PER_W // CH
NV = D // 16


def _sc_body(x_hbm, t_hbm, o_hbm, xb, tb, sem_x, sem_t):
    wid = lax.axis_index("s") * NC + lax.axis_index("c")
    base = wid * S_PER_W

    def chunk_body(ci, carry):
        s0 = base + ci * CH
        cx = pltpu.make_async_copy(x_hbm.at[pl.ds(s0, CH)], xb, sem_x)
        ct = pltpu.make_async_copy(t_hbm.at[pl.ds(s0, CH)], tb, sem_t)
        cx.start()
        ct.start()
        cx.wait()
        ct.wait()

        def pos_body(p, c2):
            for v in range(NV):
                tv = tb[p, pl.ds(v * 16, 16)]
                for b in range(B):
                    xb[p, b, pl.ds(v * 16, 16)] += tv
            return c2

        lax.fori_loop(0, CH, pos_body, 0)
        pltpu.sync_copy(xb, o_hbm.at[pl.ds(s0, CH)])
        return carry

    lax.fori_loop(0, NCHUNK, chunk_body, 0)


def kernel(x, table):
    mesh = plsc.VectorSubcoreMesh(core_axis_name="c", subcore_axis_name="s")
    f = functools.partial(
        pl.kernel,
        mesh=mesh,
        out_type=jax.ShapeDtypeStruct((S, B, D), jnp.float32),
        scratch_types=[
            pltpu.VMEM((CH, B, D), jnp.float32),
            pltpu.VMEM((CH, D), jnp.float32),
            pltpu.SemaphoreType.DMA,
            pltpu.SemaphoreType.DMA,
        ],
    )(_sc_body)
    return f(x, table)


# speedup vs baseline: 1.4074x; 1.4074x over previous
import functools
import jax
import jax.numpy as jnp
from jax import lax
from jax.experimental import pallas as pl
from jax.experimental.pallas import tpu as pltpu
from jax.experimental.pallas import tpu_sc as plsc

S, B, D = 4096, 4, 1024
NC, NS = 2, 16
NW = NC * NS
S_PER_W = S // NW
CH = 4
NCHUNK = S_PER_W // CH
NBUF = 4
NV = D // 16


def _sc_body(x_hbm, t_hbm, o_hbm, xb, tb,
             si0, si1, si2, si3, so0, so1, so2, so3):
    sin = (si0, si1, si2, si3)
    sout = (so0, so1, so2, so3)
    wid = lax.axis_index("s") * NC + lax.axis_index("c")
    base = wid * S_PER_W

    def start_in(ci, b):
        s0 = base + ci * CH
        pltpu.make_async_copy(x_hbm.at[pl.ds(s0, CH)], xb.at[b], sin[b]).start()
        pltpu.make_async_copy(t_hbm.at[pl.ds(s0, CH)], tb.at[b], sin[b]).start()

    def wait_in(b):
        pltpu.make_async_copy(x_hbm.at[pl.ds(0, CH)], xb.at[b], sin[b]).wait()
        pltpu.make_async_copy(t_hbm.at[pl.ds(0, CH)], tb.at[b], sin[b]).wait()

    def start_out(ci, b):
        dst = o_hbm.at[pl.ds(base + ci * CH, CH)]
        pltpu.make_async_copy(xb.at[b], dst, sout[b]).start()

    def wait_out(b):
        dst = o_hbm.at[pl.ds(base, CH)]
        pltpu.make_async_copy(xb.at[b], dst, sout[b]).wait()

    def compute(b):
        def pos_body(p, c2):
            for v in range(NV):
                tv = tb[b, p, pl.ds(v * 16, 16)]
                for bb in range(B):
                    xb[b, p, bb, pl.ds(v * 16, 16)] += tv
            return c2

        lax.fori_loop(0, CH, pos_body, 0)

    start_in(0, 0)

    def group_body(g, carry):
        for b in range(NBUF):
            ci = g * NBUF + b
            bn = (b + 1) % NBUF

            @pl.when(ci >= NBUF - 1)
            def _():
                wait_out(bn)

            @pl.when(ci + 1 < NCHUNK)
            def _():
                start_in(ci + 1, bn)

            wait_in(b)
            compute(b)
            start_out(ci, b)
        return carry

    lax.fori_loop(0, NCHUNK // NBUF, group_body, 0)
    for b in ((NCHUNK - 3) % NBUF, (NCHUNK - 2) % NBUF, (NCHUNK - 1) % NBUF):
        wait_out(b)


def kernel(x, table):
    mesh = plsc.VectorSubcoreMesh(core_axis_name="c", subcore_axis_name="s")
    f = functools.partial(
        pl.kernel,
        mesh=mesh,
        out_type=jax.ShapeDtypeStruct((S, B, D), jnp.float32),
        scratch_types=[
            pltpu.VMEM((NBUF, CH, B, D), jnp.float32),
            pltpu.VMEM((NBUF, CH, D), jnp.float32),
        ] + [pltpu.SemaphoreType.DMA] * (2 * NBUF),
    )(_sc_body)
    return f(x, table)
